# Initial kernel scaffold; baseline (speedup 1.0000x reference)
import jax
import jax.numpy as jnp
from jax.experimental import pallas as pl


def kernel(x, edge_index, bn_gamma, bn_beta, W1, b1, W2, b2, Wm1, bm1, Wm2, bm2, Wm3, bm3):
    raise NotImplementedError("write your pallas kernel here")


# speedup vs baseline: 12.2892x; 12.2892x over previous
import functools

import jax
import jax.numpy as jnp
from jax import lax
from jax.experimental import pallas as pl
from jax.experimental.pallas import tpu as pltpu
from jax.experimental.pallas import tpu_sc as plsc

N = 10000
E = 320000
D = 128
C = 40

NC = 2
NS = 16
NW = NC * NS
PER_W = E // NW
ECH = 80
N_ECH = PER_W // ECH
RCH = 80
N_RCH = N // RCH
RITER = (N_RCH + NS - 1) // NS

def _zero_buf(buf, rows, width):
    def row(i, _):
        def col(j, _):
            buf[i, pl.ds(j * 16, 16)] = jnp.zeros((16,), jnp.float32)
            return 0
        return lax.fori_loop(0, width // 16, col, 0)
    lax.fori_loop(0, rows, row, 0)


def _deg_kernel(dst_hbm, out_hbm, dst_v, ones_v, buf_v, acc_sh, sem):
    c = lax.axis_index("c")
    s = lax.axis_index("s")
    wid = c * NS + s

    def fill_ones(i, _):
        ones_v[pl.ds(i * 16, 16)] = jnp.ones((16,), jnp.float32)
        return 0
    lax.fori_loop(0, ECH // 16, fill_ones, 0)

    def fill_zeros(i, _):
        buf_v[pl.ds(i * 16, 16)] = jnp.zeros((16,), jnp.float32)
        return 0
    lax.fori_loop(0, RCH // 16, fill_zeros, 0)

    def zacc(it, _):
        j = it * NS + s
        @pl.when(j < N_RCH)
        def _():
            pltpu.sync_copy(buf_v, acc_sh.at[pl.ds(j * RCH, RCH)])
        return 0
    lax.fori_loop(0, RITER, zacc, 0)
    plsc.subcore_barrier()

    def step(i, _):
        base = wid * PER_W + i * ECH
        pltpu.sync_copy(dst_hbm.at[pl.ds(base, ECH)], dst_v)
        pltpu.sync_copy(ones_v, acc_sh.at[dst_v], add=True)
        return 0
    lax.fori_loop(0, N_ECH, step, 0)
    plsc.subcore_barrier()

    def wout(it, _):
        j = it * NS + s
        @pl.when(j < N_RCH)
        def _():
            pltpu.sync_copy(acc_sh.at[pl.ds(j * RCH, RCH)], buf_v)
            pltpu.sync_copy(buf_v, out_hbm.at[pl.ds(c * N + j * RCH, RCH)])
        return 0
    lax.fori_loop(0, RITER, wout, 0)


@functools.cache
def _deg_call():
    mesh = plsc.VectorSubcoreMesh(core_axis_name="c", subcore_axis_name="s")
    return pl.kernel(
        _deg_kernel,
        mesh=mesh,
        out_type=jax.ShapeDtypeStruct((NC * N,), jnp.float32),
        scratch_types=[
            pltpu.VMEM((ECH,), jnp.int32),
            pltpu.VMEM((ECH,), jnp.float32),
            pltpu.VMEM((RCH,), jnp.float32),
            pltpu.VMEM_SHARED((N,), jnp.float32),
            pltpu.SemaphoreType.DMA,
        ],
    )


def _segsum_kernel(g_hbm, src_hbm, dst_hbm, out_hbm, src_v, dst_v, rows_v,
                   zbuf_v, acc_sh, sem):
    c = lax.axis_index("c")
    s = lax.axis_index("s")
    wid = c * NS + s

    _zero_buf(zbuf_v, RCH, D)

    def zacc(it, _):
        j = it * NS + s
        @pl.when(j < N_RCH)
        def _():
            pltpu.sync_copy(zbuf_v, acc_sh.at[pl.ds(j * RCH, RCH)])
        return 0
    lax.fori_loop(0, RITER, zacc, 0)
    plsc.subcore_barrier()

    def step(i, _):
        base = wid * PER_W + i * ECH
        pltpu.sync_copy(src_hbm.at[pl.ds(base, ECH)], src_v)
        pltpu.sync_copy(dst_hbm.at[pl.ds(base, ECH)], dst_v)
        pltpu.async_copy(g_hbm.at[src_v], rows_v, sem).wait()
        pltpu.sync_copy(rows_v, acc_sh.at[dst_v], add=True)
        return 0
    lax.fori_loop(0, N_ECH, step, 0)
    plsc.subcore_barrier()

    def wout(it, _):
        j = it * NS + s
        @pl.when(j < N_RCH)
        def _():
            pltpu.sync_copy(acc_sh.at[pl.ds(j * RCH, RCH)], rows_v)
            pltpu.sync_copy(rows_v, out_hbm.at[pl.ds(c * N + j * RCH, RCH)])
        return 0
    lax.fori_loop(0, RITER, wout, 0)


@functools.cache
def _segsum_call():
    mesh = plsc.VectorSubcoreMesh(core_axis_name="c", subcore_axis_name="s")
    return pl.kernel(
        _segsum_kernel,
        mesh=mesh,
        out_type=jax.ShapeDtypeStruct((NC * N, D), jnp.float32),
        scratch_types=[
            pltpu.VMEM((ECH,), jnp.int32),
            pltpu.VMEM((ECH,), jnp.int32),
            pltpu.VMEM((RCH, D), jnp.float32),
            pltpu.VMEM((RCH, D), jnp.float32),
            pltpu.VMEM_SHARED((N, D), jnp.float32),
            pltpu.SemaphoreType.DMA,
        ],
    )


def _bn_mm_body(x_ref, gam_ref, bet_ref, w_ref, deg_ref, g_out, dinv_out):
    x = x_ref[...]
    mean = jnp.mean(x, axis=0, keepdims=True)
    var = jnp.mean((x - mean) ** 2, axis=0, keepdims=True)
    xn = (x - mean) * lax.rsqrt(var + 1e-5) * gam_ref[...] + bet_ref[...]
    deg = deg_ref[:, 0:1] + deg_ref[:, 1:2] + 1.0
    dinv = lax.rsqrt(deg)
    g_out[...] = jnp.dot(xn, w_ref[...], preferred_element_type=jnp.float32) * dinv
    dinv_out[...] = dinv


def _update_mm_body(parts_ref, g_ref, dinv_ref, b_ref, w_ref, out_ref):
    dinv = dinv_ref[...]
    agg = parts_ref[0:N] + parts_ref[N:2 * N] + g_ref[...]
    h = jax.nn.relu(dinv * agg + b_ref[...])
    out_ref[...] = jnp.dot(h, w_ref[...], preferred_element_type=jnp.float32) * dinv


def _final_body(parts_ref, g_ref, dinv_ref, b_ref, wm1_ref, bm1_ref, wm2_ref,
                bm2_ref, wm3_ref, bm3_ref, out_ref):
    dinv = dinv_ref[...]
    agg = parts_ref[0:N] + parts_ref[N:2 * N] + g_ref[...]
    h = jax.nn.relu(dinv * agg + b_ref[...])
    h = jnp.dot(h, wm1_ref[...], preferred_element_type=jnp.float32) + bm1_ref[...]
    h = jnp.dot(h, wm2_ref[...], preferred_element_type=jnp.float32) + bm2_ref[...]
    h = jnp.dot(h, wm3_ref[...], preferred_element_type=jnp.float32) + bm3_ref[...]
    m = jnp.max(h, axis=1, keepdims=True)
    z = h - m
    lse = jnp.log(jnp.sum(jnp.exp(z), axis=1, keepdims=True))
    out_ref[...] = z - lse


def kernel(x, edge_index, bn_gamma, bn_beta, W1, b1, W2, b2, Wm1, bm1, Wm2,
           bm2, Wm3, bm3):
    src = edge_index[0].astype(jnp.int32)
    dst = edge_index[1].astype(jnp.int32)

    deg_parts = _deg_call()(dst).reshape(2, N).transpose(1, 0)

    g1, dinv = pl.pallas_call(
        _bn_mm_body,
        out_shape=(
            jax.ShapeDtypeStruct((N, D), jnp.float32),
            jax.ShapeDtypeStruct((N, 1), jnp.float32),
        ),
    )(x, bn_gamma.reshape(1, D), bn_beta.reshape(1, D), W1, deg_parts)

    agg1 = _segsum_call()(g1, src, dst)

    g2 = pl.pallas_call(
        _update_mm_body,
        out_shape=jax.ShapeDtypeStruct((N, D), jnp.float32),
    )(agg1, g1, dinv, b1.reshape(1, D), W2)

    agg2 = _segsum_call()(g2, src, dst)

    out = pl.pallas_call(
        _final_body,
        out_shape=jax.ShapeDtypeStruct((N, C), jnp.float32),
    )(agg2, g2, dinv, b2.reshape(1, D), Wm1, bm1.reshape(1, D), Wm2,
      bm2.reshape(1, D), Wm3, bm3.reshape(1, C))
    return out


# speedup vs baseline: 31.1334x; 2.5334x over previous
import functools

import jax
import jax.numpy as jnp
from jax import lax
from jax.experimental import pallas as pl
from jax.experimental.pallas import tpu as pltpu
from jax.experimental.pallas import tpu_sc as plsc

N = 10000
E = 320000
D = 128
C = 40

NC = 2
NS = 16
NW = NC * NS
PER_W = E // NW
ECH = 80
N_ECH = PER_W // ECH
RCH = 80
N_RCH = N // RCH
RITER = (N_RCH + NS - 1) // NS

def _zero_buf(buf, rows, width):
    def row(i, _):
        def col(j, _):
            buf[i, pl.ds(j * 16, 16)] = jnp.zeros((16,), jnp.float32)
            return 0
        return lax.fori_loop(0, width // 16, col, 0)
    lax.fori_loop(0, rows, row, 0)


def _deg_kernel(dst_hbm, out_hbm, didx0, didx1, didx2, didx3, ones_v, buf_v,
                acc_sh, dsem0, dsem1, dsem2, dsem3, ssem0, ssem1, ssem2,
                ssem3):
    c = lax.axis_index("c")
    s = lax.axis_index("s")
    wid = c * NS + s
    ebase = wid * PER_W

    def fill_ones(i, _):
        ones_v[pl.ds(i * 16, 16)] = jnp.ones((16,), jnp.float32)
        return 0
    lax.fori_loop(0, ECH // 16, fill_ones, 0)

    def fill_zeros(i, _):
        buf_v[pl.ds(i * 16, 16)] = jnp.zeros((16,), jnp.float32)
        return 0
    lax.fori_loop(0, RCH // 16, fill_zeros, 0)

    def zacc(it, _):
        j = it * NS + s
        @pl.when(j < N_RCH)
        def _():
            pltpu.sync_copy(buf_v, acc_sh.at[pl.ds(j * RCH, RCH)])
        return 0
    lax.fori_loop(0, RITER, zacc, 0)

    def d_issue(didx, dsem, i):
        pltpu.async_copy(dst_hbm.at[pl.ds(ebase + i * ECH, ECH)], didx, dsem)

    def d_wait(didx, dsem):
        pltpu.make_async_copy(
            dst_hbm.at[pl.ds(ebase, ECH)], didx, dsem).wait()

    def s_issue(didx, ssem):
        pltpu.async_copy(ones_v, acc_sh.at[didx], ssem, add=True)

    def s_wait(didx, ssem):
        pltpu.make_async_copy(ones_v, acc_sh.at[didx], ssem).wait()

    bufs = ((didx0, dsem0, ssem0), (didx1, dsem1, ssem1),
            (didx2, dsem2, ssem2), (didx3, dsem3, ssem3))
    for b in (0, 1):
        didx, dsem, _ = bufs[b]
        d_issue(didx, dsem, b)
    plsc.subcore_barrier()

    def step4(i4, _):
        for b in range(4):
            didx, dsem, ssem = bufs[b]
            pdidx, pdsem, pssem = bufs[(b + 2) % 4]
            i = i4 * 4 + b
            d_wait(didx, dsem)
            s_issue(didx, ssem)
            p = i + 2
            @pl.when(p < N_ECH)
            def _():
                @pl.when(p >= 4)
                def _():
                    s_wait(pdidx, pssem)
                d_issue(pdidx, pdsem, p)
        return 0
    lax.fori_loop(0, (N_ECH - 1) // 4, step4, 0)
    for b in (1, 2, 3):
        didx, _, ssem = bufs[b]
        s_wait(didx, ssem)
    didx, dsem, _ = bufs[0]
    d_wait(didx, dsem)
    pltpu.sync_copy(ones_v, acc_sh.at[didx], add=True)
    plsc.subcore_barrier()

    def wout(it, _):
        j = it * NS + s
        @pl.when(j < N_RCH)
        def _():
            pltpu.sync_copy(acc_sh.at[pl.ds(j * RCH, RCH)], buf_v)
            pltpu.sync_copy(buf_v, out_hbm.at[pl.ds(c * N + j * RCH, RCH)])
        return 0
    lax.fori_loop(0, RITER, wout, 0)


@functools.cache
def _deg_call():
    mesh = plsc.VectorSubcoreMesh(core_axis_name="c", subcore_axis_name="s")
    return pl.kernel(
        _deg_kernel,
        mesh=mesh,
        out_type=jax.ShapeDtypeStruct((NC * N,), jnp.float32),
        scratch_types=[
            pltpu.VMEM((ECH,), jnp.int32),
            pltpu.VMEM((ECH,), jnp.int32),
            pltpu.VMEM((ECH,), jnp.int32),
            pltpu.VMEM((ECH,), jnp.int32),
            pltpu.VMEM((ECH,), jnp.float32),
            pltpu.VMEM((RCH,), jnp.float32),
            pltpu.VMEM_SHARED((N,), jnp.float32),
        ] + [pltpu.SemaphoreType.DMA] * 8,
    )


def _segsum_kernel(g_hbm, src_hbm, dst_hbm, out_hbm, sidx0, sidx1, sidx2,
                   sidx3, didx0, didx1, didx2, didx3, rows0, rows1, rows2,
                   rows3, acc_sh, xsem0, xsem1, xsem2, xsem3, gsem0, gsem1,
                   gsem2, gsem3, dsem0, dsem1, dsem2, dsem3, ssem0, ssem1,
                   ssem2, ssem3):
    c = lax.axis_index("c")
    s = lax.axis_index("s")
    wid = c * NS + s
    ebase = wid * PER_W

    _zero_buf(rows0, RCH, D)

    def zacc(it, _):
        j = it * NS + s
        @pl.when(j < N_RCH)
        def _():
            pltpu.sync_copy(rows0, acc_sh.at[pl.ds(j * RCH, RCH)])
        return 0
    lax.fori_loop(0, RITER, zacc, 0)

    def x_issue(sidx, xsem, i):
        pltpu.async_copy(src_hbm.at[pl.ds(ebase + i * ECH, ECH)], sidx, xsem)

    def x_wait(sidx, xsem):
        pltpu.make_async_copy(
            src_hbm.at[pl.ds(ebase, ECH)], sidx, xsem).wait()

    def g_issue(sidx, rows, gsem):
        pltpu.async_copy(g_hbm.at[sidx], rows, gsem)

    def g_wait(sidx, rows, gsem):
        pltpu.make_async_copy(g_hbm.at[sidx], rows, gsem).wait()

    def d_issue(didx, dsem, i):
        pltpu.async_copy(dst_hbm.at[pl.ds(ebase + i * ECH, ECH)], didx, dsem)

    def d_wait(didx, dsem):
        pltpu.make_async_copy(
            dst_hbm.at[pl.ds(ebase, ECH)], didx, dsem).wait()

    def s_issue(rows, didx, ssem):
        pltpu.async_copy(rows, acc_sh.at[didx], ssem, add=True)

    def s_wait(rows, didx, ssem):
        pltpu.make_async_copy(rows, acc_sh.at[didx], ssem).wait()

    bufs = ((sidx0, xsem0, rows0, gsem0, didx0, dsem0, ssem0),
            (sidx1, xsem1, rows1, gsem1, didx1, dsem1, ssem1),
            (sidx2, xsem2, rows2, gsem2, didx2, dsem2, ssem2),
            (sidx3, xsem3, rows3, gsem3, didx3, dsem3, ssem3))
    for b in (0, 1, 2):
        sidx, xsem = bufs[b][0], bufs[b][1]
        x_issue(sidx, xsem, b)
    for b in (0, 1):
        sidx, xsem, rows, gsem, didx, dsem, _ = bufs[b]
        x_wait(sidx, xsem)
        g_issue(sidx, rows, gsem)
        d_issue(didx, dsem, b)
    plsc.subcore_barrier()

    def step4(i4, _):
        for b in range(4):
            sidx, xsem, rows, gsem, didx, dsem, ssem = bufs[b]
            pb = bufs[(b + 2) % 4]
            qb = bufs[(b + 3) % 4]
            i = i4 * 4 + b
            g_wait(sidx, rows, gsem)
            d_wait(didx, dsem)
            s_issue(rows, didx, ssem)
            p = i + 2
            @pl.when(p < N_ECH)
            def _():
                psidx, pxsem, prows, pgsem, pdidx, pdsem, pssem = pb
                @pl.when(p >= 4)
                def _():
                    s_wait(prows, pdidx, pssem)
                x_wait(psidx, pxsem)
                g_issue(psidx, prows, pgsem)
                d_issue(pdidx, pdsem, p)
            q = i + 3
            @pl.when(q < N_ECH)
            def _():
                x_issue(qb[0], qb[1], q)
        return 0
    lax.fori_loop(0, (N_ECH - 1) // 4, step4, 0)
    for b in (1, 2, 3):
        _, _, rows, _, didx, _, ssem = bufs[b]
        s_wait(rows, didx, ssem)
    sidx, xsem, rows, gsem, didx, dsem, _ = bufs[0]
    g_wait(sidx, rows, gsem)
    d_wait(didx, dsem)
    pltpu.sync_copy(rows, acc_sh.at[didx], add=True)
    plsc.subcore_barrier()

    def wout(it, _):
        j = it * NS + s
        @pl.when(j < N_RCH)
        def _():
            pltpu.sync_copy(acc_sh.at[pl.ds(j * RCH, RCH)], rows0)
            pltpu.sync_copy(rows0, out_hbm.at[pl.ds(c * N + j * RCH, RCH)])
        return 0
    lax.fori_loop(0, RITER, wout, 0)


@functools.cache
def _segsum_call():
    mesh = plsc.VectorSubcoreMesh(core_axis_name="c", subcore_axis_name="s")
    return pl.kernel(
        _segsum_kernel,
        mesh=mesh,
        out_type=jax.ShapeDtypeStruct((NC * N, D), jnp.float32),
        scratch_types=(
            [pltpu.VMEM((ECH,), jnp.int32)] * 8
            + [pltpu.VMEM((ECH, D), jnp.float32)] * 4
            + [pltpu.VMEM_SHARED((N, D), jnp.float32)]
            + [pltpu.SemaphoreType.DMA] * 16
        ),
    )


def _bn_mm_body(x_ref, gam_ref, bet_ref, w_ref, deg_ref, g_out, dinv_out):
    x = x_ref[...]
    mean = jnp.mean(x, axis=0, keepdims=True)
    var = jnp.mean((x - mean) ** 2, axis=0, keepdims=True)
    xn = (x - mean) * lax.rsqrt(var + 1e-5) * gam_ref[...] + bet_ref[...]
    deg = deg_ref[:, 0:1] + deg_ref[:, 1:2] + 1.0
    dinv = lax.rsqrt(deg)
    g_out[...] = jnp.dot(xn, w_ref[...], preferred_element_type=jnp.float32) * dinv
    dinv_out[...] = dinv


def _update_mm_body(parts_ref, g_ref, dinv_ref, b_ref, w_ref, out_ref):
    dinv = dinv_ref[...]
    agg = parts_ref[0:N] + parts_ref[N:2 * N] + g_ref[...]
    h = jax.nn.relu(dinv * agg + b_ref[...])
    out_ref[...] = jnp.dot(h, w_ref[...], preferred_element_type=jnp.float32) * dinv


def _final_body(parts_ref, g_ref, dinv_ref, b_ref, wm1_ref, bm1_ref, wm2_ref,
                bm2_ref, wm3_ref, bm3_ref, out_ref):
    dinv = dinv_ref[...]
    agg = parts_ref[0:N] + parts_ref[N:2 * N] + g_ref[...]
    h = jax.nn.relu(dinv * agg + b_ref[...])
    h = jnp.dot(h, wm1_ref[...], preferred_element_type=jnp.float32) + bm1_ref[...]
    h = jnp.dot(h, wm2_ref[...], preferred_element_type=jnp.float32) + bm2_ref[...]
    h = jnp.dot(h, wm3_ref[...], preferred_element_type=jnp.float32) + bm3_ref[...]
    m = jnp.max(h, axis=1, keepdims=True)
    z = h - m
    lse = jnp.log(jnp.sum(jnp.exp(z), axis=1, keepdims=True))
    out_ref[...] = z - lse


def kernel(x, edge_index, bn_gamma, bn_beta, W1, b1, W2, b2, Wm1, bm1, Wm2,
           bm2, Wm3, bm3):
    src = edge_index[0].astype(jnp.int32)
    dst = edge_index[1].astype(jnp.int32)

    deg_parts = _deg_call()(dst).reshape(2, N).transpose(1, 0)

    g1, dinv = pl.pallas_call(
        _bn_mm_body,
        out_shape=(
            jax.ShapeDtypeStruct((N, D), jnp.float32),
            jax.ShapeDtypeStruct((N, 1), jnp.float32),
        ),
    )(x, bn_gamma.reshape(1, D), bn_beta.reshape(1, D), W1, deg_parts)

    agg1 = _segsum_call()(g1, src, dst)

    g2 = pl.pallas_call(
        _update_mm_body,
        out_shape=jax.ShapeDtypeStruct((N, D), jnp.float32),
    )(agg1, g1, dinv, b1.reshape(1, D), W2)

    agg2 = _segsum_call()(g2, src, dst)

    out = pl.pallas_call(
        _final_body,
        out_shape=jax.ShapeDtypeStruct((N, C), jnp.float32),
    )(agg2, g2, dinv, b2.reshape(1, D), Wm1, bm1.reshape(1, D), Wm2,
      bm2.reshape(1, D), Wm3, bm3.reshape(1, C))
    return out
